# baseline (speedup 1.0000x reference)
import functools

import jax
import jax.numpy as jnp
from jax import lax
from jax.experimental import pallas as pl
from jax.experimental.pallas import tpu as pltpu
from jax.experimental.pallas import tpu_sc as plsc

_V = 1000
_VP = 1024
_B = 16384
_NC = 2
_NS = 16
_NW = _NC * _NS
_CHUNK = 32
_NCHUNK = _B // _NW // _CHUNK
_LANES = 16
_NG = 4
_GW = 128


def _lse_body(embed_ref, lse_ref):
    e = embed_ref[...]
    m = jnp.max(e, axis=1)
    s = jnp.sum(jnp.exp(e - m[:, None]), axis=1)
    lse_ref[...] = m + jnp.log(s)


def _loss_body(part_ref, loss_ref):
    loss_ref[...] = (jnp.sum(part_ref[...]) / _B).reshape(1, 1)


_TAIL0 = (_V // 128) * 128


def _gather_body(x_hbm, emb_hbm, out_hbm,
                 x_v, idx0_v, idx1_v, rows0_v, rows1_v,
                 gsem0, gsem1, ssem0, ssem1):
    cid = lax.axis_index("c")
    sid = lax.axis_index("s")
    wid = sid * _NC + cid
    base = wid * (_NCHUNK * _CHUNK)

    pltpu.sync_copy(x_hbm.at[wid], x_v)

    rows = (rows0_v, rows1_v)
    idxs = (idx0_v, idx1_v)
    gsems = (gsem0, gsem1)
    ssems = (ssem0, ssem1)
    gops = {}
    sops = {}

    def start_gather(c):
        b = c % 2
        j, k = divmod(c, _GW // _CHUNK)
        idxs[b][pl.ds(0, _LANES)] = x_v[j, pl.ds(k * _CHUNK, _LANES)]
        idxs[b][pl.ds(_LANES, _LANES)] = x_v[j, pl.ds(k * _CHUNK + _LANES,
                                                      _LANES)]
        gops[c] = pltpu.async_copy(emb_hbm.at[idxs[b]], rows[b], gsems[b])

    def start_scatter(c):
        b = c % 2
        r0 = base + c * _CHUNK
        sops[c] = (
            pltpu.async_copy(
                rows[b], out_hbm.at[pl.ds(r0, _CHUNK)], ssems[b]),
        )

    start_gather(0)
    for c in range(_NCHUNK):
        if c + 1 < _NCHUNK:
            if c >= 1:
                for op in sops[c - 1]:
                    op.wait()
            start_gather(c + 1)
        gops[c].wait()
        start_scatter(c)

    for c in (_NCHUNK - 2, _NCHUNK - 1):
        for op in sops[c]:
            op.wait()


_BB = 512


def _mmt_body(x_ref, iot_ref, emb_ref, out_ref):
    oh = (iot_ref[...] == x_ref[0]).astype(jnp.bfloat16)
    out_ref[...] = lax.dot_general(
        emb_ref[...], oh, (((0,), (0,)), ((), ())),
        preferred_element_type=jnp.float32)


def _loss_gather_body(fidx_hbm, x4_hbm, eflat_hbm, lse_hbm, part_hbm,
                      fidx_v, x4_v, pts_v, lsex_v, acc_v, psem):
    cid = lax.axis_index("c")
    sid = lax.axis_index("s")
    wid = sid * _NC + cid

    pltpu.sync_copy(fidx_hbm.at[wid], fidx_v)
    pltpu.sync_copy(x4_hbm.at[wid], x4_v)

    pops = []
    for j in range(_NG):
        pops.append(pltpu.async_copy(
            eflat_hbm.at[fidx_v.at[j]], pts_v.at[j], psem))
        pops.append(pltpu.async_copy(
            lse_hbm.at[x4_v.at[j]], lsex_v.at[j], psem))
    for op in pops:
        op.wait()

    acc = jnp.zeros((_LANES,), jnp.float32)
    for j in range(_NG):
        for g in range(_GW // _LANES):
            pts = pts_v[j, pl.ds(g * _LANES, _LANES)]
            lsx = lsex_v[j, pl.ds(g * _LANES, _LANES)]
            acc = acc + (lsx - pts)

    acc_v[...] = acc
    pltpu.sync_copy(acc_v, part_hbm.at[wid])


_CALL_CACHE = {}


def _get_calls():
    if not _CALL_CACHE:
        mesh = plsc.VectorSubcoreMesh(core_axis_name="c",
                                      subcore_axis_name="s")
        _CALL_CACHE["gather"] = functools.partial(
            pl.kernel,
            out_type=jax.ShapeDtypeStruct((_B, _VP), jnp.float32),
            mesh=mesh,
            scratch_types=[
                pltpu.VMEM((_NG, _GW), jnp.int32),
                pltpu.VMEM((_CHUNK,), jnp.int32),
                pltpu.VMEM((_CHUNK,), jnp.int32),
                pltpu.VMEM((_CHUNK, _VP), jnp.float32),
                pltpu.VMEM((_CHUNK, _VP), jnp.float32),
                pltpu.SemaphoreType.DMA,
                pltpu.SemaphoreType.DMA,
                pltpu.SemaphoreType.DMA,
                pltpu.SemaphoreType.DMA,
            ],
        )(_gather_body)
        _CALL_CACHE["loss"] = functools.partial(
            pl.kernel,
            out_type=jax.ShapeDtypeStruct((_NW, _LANES), jnp.float32),
            mesh=mesh,
            compiler_params=pltpu.CompilerParams(use_tc_tiling_on_sc=False,
                                                 needs_layout_passes=False),
            scratch_types=[
                pltpu.VMEM((_NG, _GW), jnp.int32),
                pltpu.VMEM((_NG, _GW), jnp.int32),
                pltpu.VMEM((_NG, _GW), jnp.float32),
                pltpu.VMEM((_NG, _GW), jnp.float32),
                pltpu.VMEM((_LANES,), jnp.float32),
                pltpu.SemaphoreType.DMA,
            ],
        )(_loss_gather_body)
    return _CALL_CACHE


def kernel(x, targets, embed):
    x = x.astype(jnp.int32)
    targets = targets.astype(jnp.int32)

    lse = pl.pallas_call(
        _lse_body,
        out_shape=jax.ShapeDtypeStruct((_V,), jnp.float32),
    )(embed)

    calls = _get_calls()

    embed_bf = embed.astype(jnp.bfloat16)
    x3m = x.astype(jnp.int16).reshape(_B // _BB, 1, _BB)
    iot16 = lax.broadcasted_iota(jnp.int16, (_V, _BB), 0)
    pt = pl.pallas_call(
        _mmt_body,
        out_shape=jax.ShapeDtypeStruct((_V, _B), jnp.float32),
        grid=(_B // _BB,),
        in_specs=[
            pl.BlockSpec((1, 1, _BB), lambda i: (i, 0, 0)),
            pl.BlockSpec((_V, _BB), lambda i: (0, 0)),
            pl.BlockSpec((_V, _V), lambda i: (0, 0)),
        ],
        out_specs=pl.BlockSpec((_V, _BB), lambda i: (0, i)),
    )(x3m, iot16, embed_bf)
    logits = pt.T

    x4 = x.reshape(_NW, _NG, _GW)
    fidx = (x * _V + targets).reshape(_NW, _NG, _GW)
    eflat = embed.reshape(-1)
    partials = calls["loss"](fidx, x4, eflat, lse)

    loss = pl.pallas_call(
        _loss_body,
        out_shape=jax.ShapeDtypeStruct((1, 1), jnp.float32),
    )(partials)[0, 0]

    return logits, loss


# speedup vs baseline: 1.0469x; 1.0469x over previous
import functools

import jax
import jax.numpy as jnp
from jax import lax
from jax.experimental import pallas as pl
from jax.experimental.pallas import tpu as pltpu
from jax.experimental.pallas import tpu_sc as plsc

_V = 1000
_VP = 1024
_B = 16384
_NC = 2
_NS = 16
_NW = _NC * _NS
_CHUNK = 32
_NCHUNK = _B // _NW // _CHUNK
_LANES = 16
_NG = 4
_GW = 128


def _lse_body(embed_ref, lse_ref):
    e = embed_ref[...]
    m = jnp.max(e, axis=1)
    s = jnp.sum(jnp.exp(e - m[:, None]), axis=1)
    lse_ref[...] = m + jnp.log(s)


def _loss_body(part_ref, loss_ref):
    loss_ref[...] = (jnp.sum(part_ref[...]) / _B).reshape(1, 1)


_TAIL0 = (_V // 128) * 128


def _gather_body(x_hbm, emb_hbm, out_hbm,
                 x_v, idx0_v, idx1_v, rows0_v, rows1_v,
                 gsem0, gsem1, ssem0, ssem1):
    cid = lax.axis_index("c")
    sid = lax.axis_index("s")
    wid = sid * _NC + cid
    base = wid * (_NCHUNK * _CHUNK)

    pltpu.sync_copy(x_hbm.at[wid], x_v)

    rows = (rows0_v, rows1_v)
    idxs = (idx0_v, idx1_v)
    gsems = (gsem0, gsem1)
    ssems = (ssem0, ssem1)
    gops = {}
    sops = {}

    def start_gather(c):
        b = c % 2
        j, k = divmod(c, _GW // _CHUNK)
        idxs[b][pl.ds(0, _LANES)] = x_v[j, pl.ds(k * _CHUNK, _LANES)]
        idxs[b][pl.ds(_LANES, _LANES)] = x_v[j, pl.ds(k * _CHUNK + _LANES,
                                                      _LANES)]
        gops[c] = pltpu.async_copy(emb_hbm.at[idxs[b]], rows[b], gsems[b])

    def start_scatter(c):
        b = c % 2
        r0 = base + c * _CHUNK
        sops[c] = (
            pltpu.async_copy(
                rows[b], out_hbm.at[pl.ds(r0, _CHUNK)], ssems[b]),
        )

    start_gather(0)
    for c in range(_NCHUNK):
        if c + 1 < _NCHUNK:
            if c >= 1:
                for op in sops[c - 1]:
                    op.wait()
            start_gather(c + 1)
        gops[c].wait()
        start_scatter(c)

    for c in (_NCHUNK - 2, _NCHUNK - 1):
        for op in sops[c]:
            op.wait()


_BB = 512


def _mmt_body(x_ref, emb_ref, out_ref):
    iot = lax.broadcasted_iota(jnp.int32, (_V, _BB), 0)
    oh = (iot == x_ref[0]).astype(jnp.bfloat16)
    out_ref[...] = lax.dot_general(
        emb_ref[...], oh, (((0,), (0,)), ((), ())),
        preferred_element_type=jnp.float32)


def _loss_gather_body(fidx_hbm, x4_hbm, eflat_hbm, lse_hbm, part_hbm,
                      fidx_v, x4_v, pts_v, lsex_v, acc_v, psem):
    cid = lax.axis_index("c")
    sid = lax.axis_index("s")
    wid = sid * _NC + cid

    pltpu.sync_copy(fidx_hbm.at[wid], fidx_v)
    pltpu.sync_copy(x4_hbm.at[wid], x4_v)

    pops = []
    for j in range(_NG):
        pops.append(pltpu.async_copy(
            eflat_hbm.at[fidx_v.at[j]], pts_v.at[j], psem))
        pops.append(pltpu.async_copy(
            lse_hbm.at[x4_v.at[j]], lsex_v.at[j], psem))
    for op in pops:
        op.wait()

    acc = jnp.zeros((_LANES,), jnp.float32)
    for j in range(_NG):
        for g in range(_GW // _LANES):
            pts = pts_v[j, pl.ds(g * _LANES, _LANES)]
            lsx = lsex_v[j, pl.ds(g * _LANES, _LANES)]
            acc = acc + (lsx - pts)

    acc_v[...] = acc
    pltpu.sync_copy(acc_v, part_hbm.at[wid])


_CALL_CACHE = {}


def _get_calls():
    if not _CALL_CACHE:
        mesh = plsc.VectorSubcoreMesh(core_axis_name="c",
                                      subcore_axis_name="s")
        _CALL_CACHE["gather"] = functools.partial(
            pl.kernel,
            out_type=jax.ShapeDtypeStruct((_B, _VP), jnp.float32),
            mesh=mesh,
            scratch_types=[
                pltpu.VMEM((_NG, _GW), jnp.int32),
                pltpu.VMEM((_CHUNK,), jnp.int32),
                pltpu.VMEM((_CHUNK,), jnp.int32),
                pltpu.VMEM((_CHUNK, _VP), jnp.float32),
                pltpu.VMEM((_CHUNK, _VP), jnp.float32),
                pltpu.SemaphoreType.DMA,
                pltpu.SemaphoreType.DMA,
                pltpu.SemaphoreType.DMA,
                pltpu.SemaphoreType.DMA,
            ],
        )(_gather_body)
        _CALL_CACHE["loss"] = functools.partial(
            pl.kernel,
            out_type=jax.ShapeDtypeStruct((_NW, _LANES), jnp.float32),
            mesh=mesh,
            compiler_params=pltpu.CompilerParams(use_tc_tiling_on_sc=False,
                                                 needs_layout_passes=False),
            scratch_types=[
                pltpu.VMEM((_NG, _GW), jnp.int32),
                pltpu.VMEM((_NG, _GW), jnp.int32),
                pltpu.VMEM((_NG, _GW), jnp.float32),
                pltpu.VMEM((_NG, _GW), jnp.float32),
                pltpu.VMEM((_LANES,), jnp.float32),
                pltpu.SemaphoreType.DMA,
            ],
        )(_loss_gather_body)
    return _CALL_CACHE


def kernel(x, targets, embed):
    x = x.astype(jnp.int32)
    targets = targets.astype(jnp.int32)

    lse = pl.pallas_call(
        _lse_body,
        out_shape=jax.ShapeDtypeStruct((_V,), jnp.float32),
    )(embed)

    calls = _get_calls()

    embed_bf = embed.astype(jnp.bfloat16)
    x3m = x.reshape(_B // _BB, 1, _BB)
    pt = pl.pallas_call(
        _mmt_body,
        out_shape=jax.ShapeDtypeStruct((_V, _B), jnp.float32),
        grid=(_B // _BB,),
        in_specs=[
            pl.BlockSpec((1, 1, _BB), lambda i: (i, 0, 0)),
            pl.BlockSpec((_V, _V), lambda i: (0, 0)),
        ],
        out_specs=pl.BlockSpec((_V, _BB), lambda i: (0, i)),
    )(x3m, embed_bf)
    logits = pt.T

    x4 = x.reshape(_NW, _NG, _GW)
    fidx = (x * _V + targets).reshape(_NW, _NG, _GW)
    eflat = embed.reshape(-1)
    partials = calls["loss"](fidx, x4, eflat, lse)

    loss = pl.pallas_call(
        _loss_body,
        out_shape=jax.ShapeDtypeStruct((1, 1), jnp.float32),
    )(partials)[0, 0]

    return logits, loss


# speedup vs baseline: 1.0965x; 1.0474x over previous
import functools

import jax
import jax.numpy as jnp
from jax import lax
from jax.experimental import pallas as pl
from jax.experimental.pallas import tpu as pltpu
from jax.experimental.pallas import tpu_sc as plsc

_V = 1000
_VP = 1024
_B = 16384
_NC = 2
_NS = 16
_NW = _NC * _NS
_CHUNK = 32
_NCHUNK = _B // _NW // _CHUNK
_LANES = 16
_NG = 4
_GW = 128


def _lse_body(embed_ref, lse_ref):
    e = embed_ref[...]
    m = jnp.max(e, axis=1)
    s = jnp.sum(jnp.exp(e - m[:, None]), axis=1)
    lse_ref[...] = m + jnp.log(s)


def _loss_body(part_ref, loss_ref):
    loss_ref[...] = (jnp.sum(part_ref[...]) / _B).reshape(1, 1)


_TAIL0 = (_V // 128) * 128


def _gather_body(x_hbm, emb_hbm, out_hbm,
                 x_v, idx0_v, idx1_v, rows0_v, rows1_v,
                 gsem0, gsem1, ssem0, ssem1):
    cid = lax.axis_index("c")
    sid = lax.axis_index("s")
    wid = sid * _NC + cid
    base = wid * (_NCHUNK * _CHUNK)

    pltpu.sync_copy(x_hbm.at[wid], x_v)

    rows = (rows0_v, rows1_v)
    idxs = (idx0_v, idx1_v)
    gsems = (gsem0, gsem1)
    ssems = (ssem0, ssem1)
    gops = {}
    sops = {}

    def start_gather(c):
        b = c % 2
        j, k = divmod(c, _GW // _CHUNK)
        idxs[b][pl.ds(0, _LANES)] = x_v[j, pl.ds(k * _CHUNK, _LANES)]
        idxs[b][pl.ds(_LANES, _LANES)] = x_v[j, pl.ds(k * _CHUNK + _LANES,
                                                      _LANES)]
        gops[c] = pltpu.async_copy(emb_hbm.at[idxs[b]], rows[b], gsems[b])

    def start_scatter(c):
        b = c % 2
        r0 = base + c * _CHUNK
        sops[c] = (
            pltpu.async_copy(
                rows[b], out_hbm.at[pl.ds(r0, _CHUNK)], ssems[b]),
        )

    start_gather(0)
    for c in range(_NCHUNK):
        if c + 1 < _NCHUNK:
            if c >= 1:
                for op in sops[c - 1]:
                    op.wait()
            start_gather(c + 1)
        gops[c].wait()
        start_scatter(c)

    for c in (_NCHUNK - 2, _NCHUNK - 1):
        for op in sops[c]:
            op.wait()


_BB = 1024


def _mmt_body(x_ref, emb_ref, out_ref):
    iot = lax.broadcasted_iota(jnp.int32, (_V, _BB), 0)
    oh = (iot == x_ref[0]).astype(jnp.bfloat16)
    out_ref[...] = lax.dot_general(
        emb_ref[...], oh, (((0,), (0,)), ((), ())),
        preferred_element_type=jnp.float32)


def _loss_gather_body(fidx_hbm, x4_hbm, eflat_hbm, lse_hbm, part_hbm,
                      fidx_v, x4_v, pts_v, lsex_v, acc_v, psem):
    cid = lax.axis_index("c")
    sid = lax.axis_index("s")
    wid = sid * _NC + cid

    pltpu.sync_copy(fidx_hbm.at[wid], fidx_v)
    pltpu.sync_copy(x4_hbm.at[wid], x4_v)

    pops = []
    for j in range(_NG):
        pops.append(pltpu.async_copy(
            eflat_hbm.at[fidx_v.at[j]], pts_v.at[j], psem))
        pops.append(pltpu.async_copy(
            lse_hbm.at[x4_v.at[j]], lsex_v.at[j], psem))
    for op in pops:
        op.wait()

    acc = jnp.zeros((_LANES,), jnp.float32)
    for j in range(_NG):
        for g in range(_GW // _LANES):
            pts = pts_v[j, pl.ds(g * _LANES, _LANES)]
            lsx = lsex_v[j, pl.ds(g * _LANES, _LANES)]
            acc = acc + (lsx - pts)

    acc_v[...] = acc
    pltpu.sync_copy(acc_v, part_hbm.at[wid])


_CALL_CACHE = {}


def _get_calls():
    if not _CALL_CACHE:
        mesh = plsc.VectorSubcoreMesh(core_axis_name="c",
                                      subcore_axis_name="s")
        _CALL_CACHE["gather"] = functools.partial(
            pl.kernel,
            out_type=jax.ShapeDtypeStruct((_B, _VP), jnp.float32),
            mesh=mesh,
            scratch_types=[
                pltpu.VMEM((_NG, _GW), jnp.int32),
                pltpu.VMEM((_CHUNK,), jnp.int32),
                pltpu.VMEM((_CHUNK,), jnp.int32),
                pltpu.VMEM((_CHUNK, _VP), jnp.float32),
                pltpu.VMEM((_CHUNK, _VP), jnp.float32),
                pltpu.SemaphoreType.DMA,
                pltpu.SemaphoreType.DMA,
                pltpu.SemaphoreType.DMA,
                pltpu.SemaphoreType.DMA,
            ],
        )(_gather_body)
        _CALL_CACHE["loss"] = functools.partial(
            pl.kernel,
            out_type=jax.ShapeDtypeStruct((_NW, _LANES), jnp.float32),
            mesh=mesh,
            compiler_params=pltpu.CompilerParams(use_tc_tiling_on_sc=False,
                                                 needs_layout_passes=False),
            scratch_types=[
                pltpu.VMEM((_NG, _GW), jnp.int32),
                pltpu.VMEM((_NG, _GW), jnp.int32),
                pltpu.VMEM((_NG, _GW), jnp.float32),
                pltpu.VMEM((_NG, _GW), jnp.float32),
                pltpu.VMEM((_LANES,), jnp.float32),
                pltpu.SemaphoreType.DMA,
            ],
        )(_loss_gather_body)
    return _CALL_CACHE


def kernel(x, targets, embed):
    x = x.astype(jnp.int32)
    targets = targets.astype(jnp.int32)

    lse = pl.pallas_call(
        _lse_body,
        out_shape=jax.ShapeDtypeStruct((_V,), jnp.float32),
    )(embed)

    calls = _get_calls()

    embed_bf = embed.astype(jnp.bfloat16)
    x3m = x.reshape(_B // _BB, 1, _BB)
    pt = pl.pallas_call(
        _mmt_body,
        out_shape=jax.ShapeDtypeStruct((_V, _B), jnp.float32),
        grid=(_B // _BB,),
        in_specs=[
            pl.BlockSpec((1, 1, _BB), lambda i: (i, 0, 0)),
            pl.BlockSpec((_V, _V), lambda i: (0, 0)),
        ],
        out_specs=pl.BlockSpec((_V, _BB), lambda i: (0, i)),
    )(x3m, embed_bf)
    logits = pt.T

    x4 = x.reshape(_NW, _NG, _GW)
    fidx = (x * _V + targets).reshape(_NW, _NG, _GW)
    eflat = embed.reshape(-1)
    partials = calls["loss"](fidx, x4, eflat, lse)

    loss = pl.pallas_call(
        _loss_body,
        out_shape=jax.ShapeDtypeStruct((1, 1), jnp.float32),
    )(partials)[0, 0]

    return logits, loss
